# baseline (speedup 1.0000x reference)
import functools

import jax
import jax.numpy as jnp
from jax import lax
from jax.experimental import pallas as pl
from jax.experimental.pallas import tpu as pltpu
from jax.experimental.pallas import tpu_sc as plsc

_L = 16
_CHUNK = 128


def _make_sc_call(batch, n_factors):
    info = plsc.get_sparse_core_info()
    nc, ns = info.num_cores, info.num_subcores
    nw = nc * ns
    b_per_w = batch // nw
    n_chunks = b_per_w // _CHUNK
    n_groups = b_per_w // _L
    groups_per_chunk = _CHUNK // _L
    mesh = plsc.VectorSubcoreMesh(core_axis_name="c", subcore_axis_name="s")

    @functools.partial(
        pl.kernel,
        out_type=jax.ShapeDtypeStruct((nw, b_per_w), jnp.float32),
        mesh=mesh,
        compiler_params=pltpu.CompilerParams(
            needs_layout_passes=False, use_tc_tiling_on_sc=False),
        scratch_types=dict(
            x_v=pltpu.VMEM((2 * b_per_w,), jnp.int32),
            uidx_v=pltpu.VMEM((n_chunks, _CHUNK), jnp.int32),
            aidx_v=pltpu.VMEM((n_chunks, _CHUNK), jnp.int32),
            uridx_v=pltpu.VMEM((n_chunks, _CHUNK), jnp.int32),
            aridx_v=pltpu.VMEM((n_chunks, _CHUNK), jnp.int32),
            ucol_v=pltpu.VMEM((b_per_w,), jnp.int32),
            acol_v=pltpu.VMEM((b_per_w,), jnp.int32),
            urows_v=pltpu.VMEM((b_per_w, n_factors), jnp.float32),
            arows_v=pltpu.VMEM((b_per_w, n_factors), jnp.float32),
            ubrows_v=pltpu.VMEM((b_per_w, _L), jnp.float32),
            abrows_v=pltpu.VMEM((b_per_w, _L), jnp.float32),
            out_v=pltpu.VMEM((b_per_w,), jnp.float32),
            sem=pltpu.SemaphoreType.DMA,
        ),
    )
    def sc_call(x_hbm, uf_hbm, af_hbm, ub_hbm, ab_hbm, out_hbm,
                x_v, uidx_v, aidx_v, uridx_v, aridx_v, ucol_v, acol_v,
                urows_v, arows_v, ubrows_v, abrows_v, out_v, sem):
        wid = lax.axis_index("s") * nc + lax.axis_index("c")

        pltpu.sync_copy(x_hbm.at[wid], x_v)

        lane = lax.iota(jnp.int32, _L)
        lane2 = lane * 2
        fifteen = jnp.full((_L,), 15, jnp.int32)
        for g in range(n_groups):
            ci, co = g // groups_per_chunk, (g % groups_per_chunk) * _L
            u16 = plsc.load_gather(x_v, [lane2 + (2 * g * _L)])
            a16 = plsc.load_gather(x_v, [lane2 + (2 * g * _L + 1)])
            uidx_v[ci, pl.ds(co, _L)] = u16
            aidx_v[ci, pl.ds(co, _L)] = a16
            uridx_v[ci, pl.ds(co, _L)] = lax.shift_right_logical(u16, 4)
            aridx_v[ci, pl.ds(co, _L)] = lax.shift_right_logical(a16, 4)
            ucol_v[pl.ds(g * _L, _L)] = lax.bitwise_and(u16, fifteen)
            acol_v[pl.ds(g * _L, _L)] = lax.bitwise_and(a16, fifteen)

        copies = []
        for i in range(n_chunks):
            copies.append(pltpu.async_copy(
                uf_hbm.at[uidx_v.at[i]],
                urows_v.at[pl.ds(i * _CHUNK, _CHUNK)], sem))
            copies.append(pltpu.async_copy(
                af_hbm.at[aidx_v.at[i]],
                arows_v.at[pl.ds(i * _CHUNK, _CHUNK)], sem))
            copies.append(pltpu.async_copy(
                ub_hbm.at[uridx_v.at[i]],
                ubrows_v.at[pl.ds(i * _CHUNK, _CHUNK)], sem))
            copies.append(pltpu.async_copy(
                ab_hbm.at[aridx_v.at[i]],
                abrows_v.at[pl.ds(i * _CHUNK, _CHUNK)], sem))
        for c in copies:
            c.wait()

        n_col_chunks = n_factors // _L

        def gbody(g, _):
            base = g * _L
            res = jnp.zeros((_L,), jnp.float32)
            for rr in range(_L):
                row = base + rr
                acc = (urows_v[row, pl.ds(0, _L)]
                       * arows_v[row, pl.ds(0, _L)])
                for c in range(1, n_col_chunks):
                    acc += (urows_v[row, pl.ds(c * _L, _L)]
                            * arows_v[row, pl.ds(c * _L, _L)])
                res = jnp.where(lane == rr, jnp.sum(acc), res)
            rows = lane + base
            bu = plsc.load_gather(ubrows_v, [rows, ucol_v[pl.ds(base, _L)]])
            ba = plsc.load_gather(abrows_v, [rows, acol_v[pl.ds(base, _L)]])
            r = res + bu + ba
            out_v[pl.ds(base, _L)] = 10.5 / (1.0 + jnp.exp(-r))
            return 0

        lax.fori_loop(0, n_groups, gbody, 0)

        pltpu.sync_copy(out_v, out_hbm.at[wid])

    return sc_call


def kernel(x, user_factors, anime_factors, user_bias, anime_bias):
    batch = x.shape[0]
    n_factors = user_factors.shape[1]
    info = plsc.get_sparse_core_info()
    nw = info.num_cores * info.num_subcores
    b_per_w = batch // nw

    xr = x.reshape(nw, 2 * b_per_w)
    ubr = user_bias.reshape(-1, _L)
    abr = anime_bias.reshape(-1, _L)

    sc_call = _make_sc_call(batch, n_factors)
    out = sc_call(xr, user_factors, anime_factors, ubr, abr)
    return out.reshape(batch, 1)


# speedup vs baseline: 1.2911x; 1.2911x over previous
import functools

import jax
import jax.numpy as jnp
from jax import lax
from jax.experimental import pallas as pl
from jax.experimental.pallas import tpu as pltpu
from jax.experimental.pallas import tpu_sc as plsc

_L = 16
_W = 128
_BW = 16
_PASSES = 4


def _make_sc_call(batch, n_factors):
    info = plsc.get_sparse_core_info()
    nc, ns = info.num_cores, info.num_subcores
    nw = nc * ns
    b_per_w = batch // nw
    rpp = b_per_w // _PASSES
    n_groups = rpp // _L
    mesh = plsc.VectorSubcoreMesh(core_axis_name="c", subcore_axis_name="s")

    @functools.partial(
        pl.kernel,
        out_type=jax.ShapeDtypeStruct((nw, b_per_w), jnp.float32),
        mesh=mesh,
        compiler_params=pltpu.CompilerParams(needs_layout_passes=False),
        scratch_types=dict(
            x_v=pltpu.VMEM((2 * b_per_w // _W, _W), jnp.int32),
            urows_v=pltpu.VMEM((rpp, n_factors), jnp.float32),
            arows_v=pltpu.VMEM((rpp, n_factors), jnp.float32),
            ub_v=pltpu.VMEM((rpp, _BW), jnp.float32),
            ab_v=pltpu.VMEM((rpp, _BW), jnp.float32),
            out_v=pltpu.VMEM((b_per_w,), jnp.float32),
            sem_u=pltpu.SemaphoreType.DMA,
            sem_a=pltpu.SemaphoreType.DMA,
            sem_b=pltpu.SemaphoreType.DMA,
        ),
    )
    def sc_call(x_hbm, uf_hbm, af_hbm, ub_hbm, ab_hbm, out_hbm,
                x_v, urows_v, arows_v, ub_v, ab_v, out_v,
                sem_u, sem_a, sem_b):
        wid = lax.axis_index("s") * nc + lax.axis_index("c")

        pltpu.sync_copy(x_hbm.at[wid], x_v)

        lane = lax.iota(jnp.int32, _L)
        n_col_chunks = n_factors // _L
        groups = rpp // (_L // 2)

        for p in range(_PASSES):
            pair_base = p * rpp * 2

            def load_v(g, pair_base=pair_base):
                flat = pair_base + g * _L
                return x_v[flat // _W, pl.ds(lax.rem(flat, _W), _L)]

            def copies(v, g, drain):
                for k in range(_L // 2):
                    i = g * (_L // 2) + k
                    u = v[2 * k]
                    a = v[2 * k + 1]
                    cs = [
                        pltpu.make_async_copy(
                            uf_hbm.at[pl.ds(u, 1), :],
                            urows_v.at[pl.ds(i, 1), :], sem_u),
                        pltpu.make_async_copy(
                            af_hbm.at[pl.ds(a, 1), :],
                            arows_v.at[pl.ds(i, 1), :], sem_a),
                        pltpu.make_async_copy(
                            ub_hbm.at[
                                pl.ds(lax.shift_right_logical(u, 4), 1), :],
                            ub_v.at[pl.ds(i, 1), :], sem_b),
                        pltpu.make_async_copy(
                            ab_hbm.at[
                                pl.ds(lax.shift_right_logical(a, 4), 1), :],
                            ab_v.at[pl.ds(i, 1), :], sem_b),
                    ]
                    for c in cs:
                        if drain:
                            c.wait()
                        else:
                            c.start()

            v0 = load_v(0)
            v1 = load_v(1)
            copies(v0, 0, drain=False)
            copies(v1, 1, drain=False)

            def step(j, carry):
                vj, vj1 = carry
                v2 = load_v(j + 2)
                copies(v2, j + 2, drain=False)
                copies(vj, j, drain=True)
                return (vj1, v2)

            va, vb = lax.fori_loop(0, groups - 2, step, (v0, v1))
            copies(va, groups - 2, drain=True)
            copies(vb, groups - 1, drain=True)

            def gbody(g, _, p=p):
                base = g * _L
                res = jnp.zeros((_L,), jnp.float32)
                for rr in range(_L):
                    row = base + rr
                    acc = (urows_v[row, pl.ds(0, _L)]
                           * arows_v[row, pl.ds(0, _L)])
                    for c in range(1, n_col_chunks):
                        acc += (urows_v[row, pl.ds(c * _L, _L)]
                                * arows_v[row, pl.ds(c * _L, _L)])
                    res = jnp.where(lane == rr, jnp.sum(acc), res)
                flat = p * rpp * 2 + base * 2
                xrow = jnp.full((_L,), flat // _W, jnp.int32)
                xcol = lax.rem(flat, _W) + 2 * lane
                uvals = plsc.load_gather(x_v, [xrow, xcol])
                avals = plsc.load_gather(x_v, [xrow, xcol + 1])
                m = jnp.full((_L,), _BW - 1, jnp.int32)
                i16 = lane + base
                bu = plsc.load_gather(ub_v, [i16, lax.bitwise_and(uvals, m)])
                ba = plsc.load_gather(ab_v, [i16, lax.bitwise_and(avals, m)])
                r = res + bu + ba
                out_v[pl.ds(p * rpp + base, _L)] = (
                    10.5 / (1.0 + jnp.exp(-r)))
                return 0

            lax.fori_loop(0, n_groups, gbody, 0)

        pltpu.sync_copy(out_v, out_hbm.at[wid])

    return sc_call


def kernel(x, user_factors, anime_factors, user_bias, anime_bias):
    batch = x.shape[0]
    n_factors = user_factors.shape[1]
    info = plsc.get_sparse_core_info()
    nw = info.num_cores * info.num_subcores
    b_per_w = batch // nw

    xr = x.reshape(nw, 2 * b_per_w // _W, _W)

    def bias16(b):
        n = b.shape[0]
        npad = (-n) % _BW
        return jnp.pad(b.reshape(-1), (0, npad)).reshape(-1, _BW)

    sc_call = _make_sc_call(batch, n_factors)
    out = sc_call(xr, user_factors, anime_factors,
                  bias16(user_bias), bias16(anime_bias))
    return out.reshape(batch, 1)


# speedup vs baseline: 1.3474x; 1.0436x over previous
import functools

import jax
import jax.numpy as jnp
from jax import lax
from jax.experimental import pallas as pl
from jax.experimental.pallas import tpu as pltpu
from jax.experimental.pallas import tpu_sc as plsc

_L = 16
_W = 128
_BW = 16
_PASSES = 4


def _make_sc_call(batch, n_factors):
    info = plsc.get_sparse_core_info()
    nc, ns = info.num_cores, info.num_subcores
    nw = nc * ns
    b_per_w = batch // nw
    rpp = b_per_w // _PASSES
    n_groups = rpp // _L
    mesh = plsc.VectorSubcoreMesh(core_axis_name="c", subcore_axis_name="s")

    @functools.partial(
        pl.kernel,
        out_type=jax.ShapeDtypeStruct((nw, b_per_w), jnp.float32),
        mesh=mesh,
        compiler_params=pltpu.CompilerParams(needs_layout_passes=False),
        scratch_types=dict(
            x_v=pltpu.VMEM((2 * b_per_w // _W, _W), jnp.int32),
            urows_v=pltpu.VMEM((rpp, n_factors), jnp.float32),
            arows_v=pltpu.VMEM((rpp, n_factors), jnp.float32),
            ub_v=pltpu.VMEM((rpp // 8, _W), jnp.float32),
            ab_v=pltpu.VMEM((rpp // 8, _W), jnp.float32),
            out_v=pltpu.VMEM((b_per_w,), jnp.float32),
            sem_u=pltpu.SemaphoreType.DMA,
            sem_a=pltpu.SemaphoreType.DMA,
            sem_b=pltpu.SemaphoreType.DMA,
        ),
    )
    def sc_call(x_hbm, uf_hbm, af_hbm, ub_hbm, ab_hbm, out_hbm,
                x_v, urows_v, arows_v, ub_v, ab_v, out_v,
                sem_u, sem_a, sem_b):
        wid = lax.axis_index("s") * nc + lax.axis_index("c")

        pltpu.sync_copy(x_hbm.at[wid], x_v)

        lane = lax.iota(jnp.int32, _L)
        n_col_chunks = n_factors // _L
        groups = rpp // (_L // 2)

        for p in range(_PASSES):
            pair_base = p * rpp * 2

            def load_v(g, pair_base=pair_base):
                flat = pair_base + g * _L
                return x_v[flat // _W, pl.ds(lax.rem(flat, _W), _L)]

            def copies(v, g, drain):
                for k in range(_L // 2):
                    i = g * (_L // 2) + k
                    u = v[2 * k]
                    a = v[2 * k + 1]
                    cs = [
                        pltpu.make_async_copy(
                            uf_hbm.at[pl.ds(u, 1), :],
                            urows_v.at[pl.ds(i, 1), :], sem_u),
                        pltpu.make_async_copy(
                            af_hbm.at[pl.ds(a, 1), :],
                            arows_v.at[pl.ds(i, 1), :], sem_a),
                        pltpu.make_async_copy(
                            ub_hbm.at[
                                pl.ds(lax.shift_right_logical(u, 7), 1),
                                pl.ds(lax.bitwise_and(
                                    lax.shift_right_logical(u, 4),
                                    jnp.int32(7)) * _BW, _BW)],
                            ub_v.at[pl.ds(i // 8, 1),
                                    pl.ds((i % 8) * _BW, _BW)], sem_b),
                        pltpu.make_async_copy(
                            ab_hbm.at[
                                pl.ds(lax.shift_right_logical(a, 7), 1),
                                pl.ds(lax.bitwise_and(
                                    lax.shift_right_logical(a, 4),
                                    jnp.int32(7)) * _BW, _BW)],
                            ab_v.at[pl.ds(i // 8, 1),
                                    pl.ds((i % 8) * _BW, _BW)], sem_b),
                    ]
                    for c in cs:
                        if drain:
                            c.wait()
                        else:
                            c.start()

            v0 = load_v(0)
            v1 = load_v(1)
            copies(v0, 0, drain=False)
            copies(v1, 1, drain=False)

            def step(j, carry):
                vj, vj1 = carry
                v2 = load_v(j + 2)
                copies(v2, j + 2, drain=False)
                copies(vj, j, drain=True)
                return (vj1, v2)

            va, vb = lax.fori_loop(0, groups - 2, step, (v0, v1))
            copies(va, groups - 2, drain=True)
            copies(vb, groups - 1, drain=True)

            def gbody(g, _, p=p):
                base = g * _L
                res = jnp.zeros((_L,), jnp.float32)
                for rr in range(_L):
                    row = base + rr
                    acc = (urows_v[row, pl.ds(0, _L)]
                           * arows_v[row, pl.ds(0, _L)])
                    for c in range(1, n_col_chunks):
                        acc += (urows_v[row, pl.ds(c * _L, _L)]
                                * arows_v[row, pl.ds(c * _L, _L)])
                    res = jnp.where(lane == rr, jnp.sum(acc), res)
                flat = p * rpp * 2 + base * 2
                xrow = jnp.full((_L,), flat // _W, jnp.int32)
                xcol = lax.rem(flat, _W) + 2 * lane
                uvals = plsc.load_gather(x_v, [xrow, xcol])
                avals = plsc.load_gather(x_v, [xrow, xcol + 1])
                m = jnp.full((_L,), _BW - 1, jnp.int32)
                i16 = lane + base
                br = lax.shift_right_logical(i16, 3)
                boff = lax.bitwise_and(i16, jnp.full((_L,), 7, jnp.int32))
                bu = plsc.load_gather(
                    ub_v, [br, boff * _BW + lax.bitwise_and(uvals, m)])
                ba = plsc.load_gather(
                    ab_v, [br, boff * _BW + lax.bitwise_and(avals, m)])
                r = res + bu + ba
                out_v[pl.ds(p * rpp + base, _L)] = (
                    10.5 / (1.0 + jnp.exp(-r)))
                return 0

            lax.fori_loop(0, n_groups, gbody, 0)

        pltpu.sync_copy(out_v, out_hbm.at[wid])

    return sc_call


def kernel(x, user_factors, anime_factors, user_bias, anime_bias):
    batch = x.shape[0]
    n_factors = user_factors.shape[1]
    info = plsc.get_sparse_core_info()
    nw = info.num_cores * info.num_subcores
    b_per_w = batch // nw

    xr = x.reshape(nw, 2 * b_per_w // _W, _W)

    def bias128(b):
        n = b.shape[0]
        npad = (-n) % _W
        return jnp.pad(b.reshape(-1), (0, npad)).reshape(-1, _W)

    sc_call = _make_sc_call(batch, n_factors)
    out = sc_call(xr, user_factors, anime_factors,
                  bias128(user_bias), bias128(anime_bias))
    return out.reshape(batch, 1)
